# Initial kernel scaffold; baseline (speedup 1.0000x reference)
import jax
import jax.numpy as jnp
from jax.experimental import pallas as pl


def kernel(loc_data, conf_data, priors, targets):
    raise NotImplementedError("write your pallas kernel here")


# speedup vs baseline: 23.5181x; 23.5181x over previous
import jax
import jax.numpy as jnp
from jax import lax
from jax.experimental import pallas as pl

_C = 21
_THR = 0.5
_V0, _V1 = 0.1, 0.2
_B, _P, _NOBJ = 32, 8732, 12


def _per_image_kernel(tgt_ref, pri_ref, loc_ref, conf_ref, vneg_ref, stats_ref):
    tgt = tgt_ref[0]
    tx1 = tgt[:, 0:1]
    ty1 = tgt[:, 1:2]
    tx2 = tgt[:, 2:3]
    ty2 = tgt[:, 3:4]
    tlab = tgt[:, 4:5]
    pcx = pri_ref[0:1, :]
    pcy = pri_ref[1:2, :]
    pw = pri_ref[2:3, :]
    ph = pri_ref[3:4, :]
    px1 = pcx - pw * 0.5
    py1 = pcy - ph * 0.5
    px2 = pcx + pw * 0.5
    py2 = pcy + ph * 0.5

    iw = jnp.clip(jnp.minimum(tx2, px2) - jnp.maximum(tx1, px1), 0.0, None)
    ih = jnp.clip(jnp.minimum(ty2, py2) - jnp.maximum(ty1, py1), 0.0, None)
    inter = iw * ih
    area_a = (tx2 - tx1) * (ty2 - ty1)
    area_b = (px2 - px1) * (py2 - py1)
    ov = inter / (area_a + area_b - inter)

    iota_t = lax.broadcasted_iota(jnp.int32, ov.shape, 0)
    iota_p = lax.broadcasted_iota(jnp.int32, ov.shape, 1)
    bto = jnp.max(ov, axis=0, keepdims=True)
    btidx = jnp.min(jnp.where(ov == bto, iota_t, _NOBJ), axis=0, keepdims=True)
    rowmax = jnp.max(ov, axis=1, keepdims=True)
    bpi = jnp.min(jnp.where(ov == rowmax, iota_p, _P), axis=1, keepdims=True)

    forced = jnp.max(jnp.where(iota_p == bpi, iota_t, -1), axis=0, keepdims=True)
    is_f = forced >= 0
    bti = jnp.where(is_f, forced, btidx)
    btov = jnp.where(is_f, 2.0, bto)

    sel = bti == iota_t

    def pick(col):
        return jnp.sum(jnp.where(sel, col, 0.0), axis=0, keepdims=True)

    lab = pick(tlab)
    conf_t = jnp.where(btov < _THR, 0, lab.astype(jnp.int32) + 1)
    pos = conf_t > 0
    posf = pos.astype(jnp.float32)
    num_pos = jnp.sum(posf)

    mx1 = pick(tx1)
    my1 = pick(ty1)
    mx2 = pick(tx2)
    my2 = pick(ty2)
    g_cx = ((mx1 + mx2) * 0.5 - pcx) / (_V0 * pw)
    g_cy = ((my1 + my2) * 0.5 - pcy) / (_V0 * ph)
    g_w = jnp.log((mx2 - mx1) / pw) / _V1
    g_h = jnp.log((my2 - my1) / ph) / _V1
    loc = loc_ref[0]

    def sl1(d):
        ad = jnp.abs(d)
        return jnp.where(ad < 1.0, 0.5 * ad * ad, ad - 0.5)

    l_terms = (sl1(loc[0:1, :] - g_cx) + sl1(loc[1:2, :] - g_cy)
               + sl1(loc[2:3, :] - g_w) + sl1(loc[3:4, :] - g_h))
    loss_l = jnp.sum(l_terms * posf)

    x = conf_ref[0]
    m = jnp.max(x, axis=0, keepdims=True)
    s = jnp.sum(jnp.exp(x - m), axis=0, keepdims=True)
    lse = jnp.log(s) + m
    iota_c = lax.broadcasted_iota(jnp.int32, x.shape, 0)
    gathered = jnp.sum(jnp.where(iota_c == conf_t, x, 0.0), axis=0, keepdims=True)
    v = lse - gathered
    pos_ce = jnp.sum(v * posf)
    vneg_ref[...] = jnp.where(pos, 0.0, v).reshape(1, 1, _P)

    lane = lax.broadcasted_iota(jnp.int32, (1, 128), 1)
    stats = jnp.where(lane == 0, num_pos,
                      jnp.where(lane == 1, loss_l,
                                jnp.where(lane == 2, pos_ce, 0.0)))
    stats_ref[...] = stats.reshape(1, 1, 128)


def _finalize_kernel(vneg_ref, stats_ref, out_l_ref, out_c_ref):
    v = vneg_ref[:, 0, :]
    stats = stats_ref[:, 0, :]
    num_pos = stats[:, 0:1]
    k = jnp.minimum(num_pos * 3.0, float(_P - 1))

    vb = lax.bitcast_convert_type(v, jnp.int32)
    prefix = jnp.zeros_like(vb[:, 0:1])
    for bit in range(30, -1, -1):
        cand = prefix | (1 << bit)
        c = jnp.sum((vb >= cand).astype(jnp.float32), axis=1, keepdims=True)
        prefix = jnp.where(c >= k, cand, prefix)
    t = lax.bitcast_convert_type(prefix, jnp.float32)
    gt = vb > prefix
    c1 = jnp.sum(gt.astype(jnp.float32), axis=1, keepdims=True)
    sum_gt = jnp.sum(jnp.where(gt, v, 0.0), axis=1, keepdims=True)
    topk = sum_gt + (k - c1) * t

    n = jnp.sum(num_pos)
    out_l_ref[...] = (jnp.sum(stats[:, 1:2]) / n).reshape(1, 1)
    out_c_ref[...] = ((jnp.sum(stats[:, 2:3]) + jnp.sum(topk)) / n).reshape(1, 1)


def kernel(loc_data, conf_data, priors, targets):
    conf_tr = jnp.transpose(conf_data, (0, 2, 1))
    loc_tr = jnp.transpose(loc_data, (0, 2, 1))
    priors_t = jnp.transpose(priors)

    vneg, stats = pl.pallas_call(
        _per_image_kernel,
        grid=(_B,),
        in_specs=[
            pl.BlockSpec((1, _NOBJ, 5), lambda b: (b, 0, 0)),
            pl.BlockSpec((4, _P), lambda b: (0, 0)),
            pl.BlockSpec((1, 4, _P), lambda b: (b, 0, 0)),
            pl.BlockSpec((1, _C, _P), lambda b: (b, 0, 0)),
        ],
        out_specs=[
            pl.BlockSpec((1, 1, _P), lambda b: (b, 0, 0)),
            pl.BlockSpec((1, 1, 128), lambda b: (b, 0, 0)),
        ],
        out_shape=[
            jax.ShapeDtypeStruct((_B, 1, _P), jnp.float32),
            jax.ShapeDtypeStruct((_B, 1, 128), jnp.float32),
        ],
    )(targets, priors_t, loc_tr, conf_tr)

    loss_l, loss_c = pl.pallas_call(
        _finalize_kernel,
        out_shape=[
            jax.ShapeDtypeStruct((1, 1), jnp.float32),
            jax.ShapeDtypeStruct((1, 1), jnp.float32),
        ],
    )(vneg, stats)
    return loss_l[0, 0], loss_c[0, 0]


# speedup vs baseline: 32.1787x; 1.3683x over previous
import jax
import jax.numpy as jnp
from jax import lax
from jax.experimental import pallas as pl
from jax.experimental.pallas import tpu as pltpu

_C = 21
_THR = 0.5
_V0, _V1 = 0.1, 0.2
_B, _P, _NOBJ = 32, 8732, 12


def _loss_kernel(tgt_ref, pri_ref, loc_ref, conf_ref, out_l_ref, out_c_ref,
                 vneg_s, np_s, acc_s):
    b = pl.program_id(0)

    @pl.when(b < _B)
    def _per_image():
        tgt = tgt_ref[0]
        tx1 = tgt[:, 0:1]
        ty1 = tgt[:, 1:2]
        tx2 = tgt[:, 2:3]
        ty2 = tgt[:, 3:4]
        tlab = tgt[:, 4:5]
        pcx = pri_ref[0:1, :]
        pcy = pri_ref[1:2, :]
        pw = pri_ref[2:3, :]
        ph = pri_ref[3:4, :]
        px1 = pcx - pw * 0.5
        py1 = pcy - ph * 0.5
        px2 = pcx + pw * 0.5
        py2 = pcy + ph * 0.5

        iw = jnp.clip(jnp.minimum(tx2, px2) - jnp.maximum(tx1, px1), 0.0, None)
        ih = jnp.clip(jnp.minimum(ty2, py2) - jnp.maximum(ty1, py1), 0.0, None)
        inter = iw * ih
        area_a = (tx2 - tx1) * (ty2 - ty1)
        area_b = (px2 - px1) * (py2 - py1)
        ov = inter / (area_a + area_b - inter)

        iota_t = lax.broadcasted_iota(jnp.int32, ov.shape, 0)
        iota_p = lax.broadcasted_iota(jnp.int32, ov.shape, 1)
        bto = jnp.max(ov, axis=0, keepdims=True)
        btidx = jnp.min(jnp.where(ov == bto, iota_t, _NOBJ), axis=0,
                        keepdims=True)
        rowmax = jnp.max(ov, axis=1, keepdims=True)
        bpi = jnp.min(jnp.where(ov == rowmax, iota_p, _P), axis=1,
                      keepdims=True)

        forced = jnp.max(jnp.where(iota_p == bpi, iota_t, -1), axis=0,
                         keepdims=True)
        is_f = forced >= 0
        bti = jnp.where(is_f, forced, btidx)
        btov = jnp.where(is_f, 2.0, bto)

        sel = bti == iota_t

        def pick(col):
            return jnp.sum(jnp.where(sel, col, 0.0), axis=0, keepdims=True)

        lab = pick(tlab)
        conf_t = jnp.where(btov < _THR, 0, lab.astype(jnp.int32) + 1)
        pos = conf_t > 0
        posf = pos.astype(jnp.float32)
        num_pos = jnp.sum(posf)

        mx1 = pick(tx1)
        my1 = pick(ty1)
        mx2 = pick(tx2)
        my2 = pick(ty2)
        g_cx = ((mx1 + mx2) * 0.5 - pcx) / (_V0 * pw)
        g_cy = ((my1 + my2) * 0.5 - pcy) / (_V0 * ph)
        g_w = jnp.log((mx2 - mx1) / pw) / _V1
        g_h = jnp.log((my2 - my1) / ph) / _V1
        loc = loc_ref[0]

        def sl1(d):
            ad = jnp.abs(d)
            return jnp.where(ad < 1.0, 0.5 * ad * ad, ad - 0.5)

        l_terms = (sl1(loc[0:1, :] - g_cx) + sl1(loc[1:2, :] - g_cy)
                   + sl1(loc[2:3, :] - g_w) + sl1(loc[3:4, :] - g_h))
        loss_l = jnp.sum(l_terms * posf)

        x = conf_ref[0]
        m = jnp.max(x, axis=0, keepdims=True)
        s = jnp.sum(jnp.exp(x - m), axis=0, keepdims=True)
        lse = jnp.log(s) + m
        iota_c = lax.broadcasted_iota(jnp.int32, x.shape, 0)
        gathered = jnp.sum(jnp.where(iota_c == conf_t, x, 0.0), axis=0,
                           keepdims=True)
        v = lse - gathered
        pos_ce = jnp.sum(v * posf)

        vneg_s[pl.ds(b, 1), :] = jnp.where(pos, 0.0, v)
        np_s[pl.ds(b, 1), :] = jnp.full((1, 128), num_pos, jnp.float32)

        @pl.when(b == 0)
        def _init():
            acc_s[0] = 0.0
            acc_s[1] = 0.0

        acc_s[0] += loss_l
        acc_s[1] += pos_ce

    @pl.when(b == _B)
    def _finalize():
        vn = vneg_s[...]
        num_pos = np_s[:, 0:1]
        k = jnp.minimum(num_pos * 3.0, float(_P - 1))

        vb = lax.bitcast_convert_type(vn, jnp.int32)
        prefix = jnp.zeros_like(vb[:, 0:1])
        for bit in range(30, -1, -1):
            cand = prefix | (1 << bit)
            c = jnp.sum((vb >= cand).astype(jnp.float32), axis=1,
                        keepdims=True)
            prefix = jnp.where(c >= k, cand, prefix)
        t = lax.bitcast_convert_type(prefix, jnp.float32)
        gt = vb > prefix
        c1 = jnp.sum(gt.astype(jnp.float32), axis=1, keepdims=True)
        sum_gt = jnp.sum(jnp.where(gt, vn, 0.0), axis=1, keepdims=True)
        topk = sum_gt + (k - c1) * t

        n = jnp.sum(num_pos)
        out_l_ref[...] = (acc_s[0] / n).reshape(1, 1)
        out_c_ref[...] = ((acc_s[1] + jnp.sum(topk)) / n).reshape(1, 1)


def kernel(loc_data, conf_data, priors, targets):
    conf_tr = jnp.transpose(conf_data, (0, 2, 1))
    loc_tr = jnp.transpose(loc_data, (0, 2, 1))
    priors_t = jnp.transpose(priors)

    loss_l, loss_c = pl.pallas_call(
        _loss_kernel,
        grid=(_B + 1,),
        in_specs=[
            pl.BlockSpec((1, _NOBJ, 5), lambda b: (jnp.minimum(b, _B - 1), 0, 0)),
            pl.BlockSpec((4, _P), lambda b: (0, 0)),
            pl.BlockSpec((1, 4, _P), lambda b: (jnp.minimum(b, _B - 1), 0, 0)),
            pl.BlockSpec((1, _C, _P), lambda b: (jnp.minimum(b, _B - 1), 0, 0)),
        ],
        out_specs=[
            pl.BlockSpec((1, 1), lambda b: (0, 0)),
            pl.BlockSpec((1, 1), lambda b: (0, 0)),
        ],
        out_shape=[
            jax.ShapeDtypeStruct((1, 1), jnp.float32),
            jax.ShapeDtypeStruct((1, 1), jnp.float32),
        ],
        scratch_shapes=[
            pltpu.VMEM((_B, _P), jnp.float32),
            pltpu.VMEM((_B, 128), jnp.float32),
            pltpu.SMEM((2,), jnp.float32),
        ],
    )(targets, priors_t, loc_tr, conf_tr)
    return loss_l[0, 0], loss_c[0, 0]


# speedup vs baseline: 39.9190x; 1.2405x over previous
import jax
import jax.numpy as jnp
from jax import lax
from jax.experimental import pallas as pl
from jax.experimental.pallas import tpu as pltpu

_C = 21
_THR = 0.5
_V0, _V1 = 0.1, 0.2
_B, _P, _NOBJ = 32, 8732, 12


def _loss_kernel(tgt_ref, pri_ref, loc_ref, conf_ref, out_l_ref, out_c_ref,
                 vneg_s, np_s, acc_s):
    b = pl.program_id(0)

    @pl.when(b < _B)
    def _per_image():
        tgt = tgt_ref[0]
        tx1 = tgt[:, 0:1]
        ty1 = tgt[:, 1:2]
        tx2 = tgt[:, 2:3]
        ty2 = tgt[:, 3:4]
        tlab = tgt[:, 4:5]
        pcx = pri_ref[0:1, :]
        pcy = pri_ref[1:2, :]
        pw = pri_ref[2:3, :]
        ph = pri_ref[3:4, :]
        px1 = pcx - pw * 0.5
        py1 = pcy - ph * 0.5
        px2 = pcx + pw * 0.5
        py2 = pcy + ph * 0.5

        iw = jnp.clip(jnp.minimum(tx2, px2) - jnp.maximum(tx1, px1), 0.0, None)
        ih = jnp.clip(jnp.minimum(ty2, py2) - jnp.maximum(ty1, py1), 0.0, None)
        inter = iw * ih
        area_a = (tx2 - tx1) * (ty2 - ty1)
        area_b = (px2 - px1) * (py2 - py1)
        ov = inter / (area_a + area_b - inter)

        iota_t = lax.broadcasted_iota(jnp.int32, ov.shape, 0)
        iota_p = lax.broadcasted_iota(jnp.int32, ov.shape, 1)
        bto = jnp.max(ov, axis=0, keepdims=True)
        btidx = jnp.min(jnp.where(ov == bto, iota_t, _NOBJ), axis=0,
                        keepdims=True)
        rowmax = jnp.max(ov, axis=1, keepdims=True)
        bpi = jnp.min(jnp.where(ov == rowmax, iota_p, _P), axis=1,
                      keepdims=True)

        forced = jnp.max(jnp.where(iota_p == bpi, iota_t, -1), axis=0,
                         keepdims=True)
        is_f = forced >= 0
        bti = jnp.where(is_f, forced, btidx)
        btov = jnp.where(is_f, 2.0, bto)

        onehot = (bti == iota_t).astype(jnp.float32)
        picked = lax.dot_general(tgt, onehot, (((0,), (0,)), ((), ())),
                                 preferred_element_type=jnp.float32)
        mx1 = picked[0:1, :]
        my1 = picked[1:2, :]
        mx2 = picked[2:3, :]
        my2 = picked[3:4, :]
        lab = picked[4:5, :]
        conf_t = jnp.where(btov < _THR, 0, lab.astype(jnp.int32) + 1)
        pos = conf_t > 0
        posf = pos.astype(jnp.float32)
        num_pos = jnp.sum(posf)
        g_cx = ((mx1 + mx2) * 0.5 - pcx) / (_V0 * pw)
        g_cy = ((my1 + my2) * 0.5 - pcy) / (_V0 * ph)
        g_w = jnp.log((mx2 - mx1) / pw) / _V1
        g_h = jnp.log((my2 - my1) / ph) / _V1
        loc = loc_ref[0]

        def sl1(d):
            ad = jnp.abs(d)
            return jnp.where(ad < 1.0, 0.5 * ad * ad, ad - 0.5)

        l_terms = (sl1(loc[0:1, :] - g_cx) + sl1(loc[1:2, :] - g_cy)
                   + sl1(loc[2:3, :] - g_w) + sl1(loc[3:4, :] - g_h))
        loss_l = jnp.sum(l_terms * posf)

        x = conf_ref[0]
        m = jnp.max(x, axis=0, keepdims=True)
        s = jnp.sum(jnp.exp(x - m), axis=0, keepdims=True)
        lse = jnp.log(s) + m
        iota_c = lax.broadcasted_iota(jnp.int32, x.shape, 0)
        gathered = jnp.sum(jnp.where(iota_c == conf_t, x, 0.0), axis=0,
                           keepdims=True)
        v = lse - gathered
        pos_ce = jnp.sum(v * posf)

        vneg_s[pl.ds(b, 1), :] = jnp.where(pos, 0.0, v)
        np_s[pl.ds(b, 1), :] = jnp.full((1, 128), num_pos, jnp.float32)

        @pl.when(b == 0)
        def _init():
            acc_s[0] = 0.0
            acc_s[1] = 0.0

        acc_s[0] += loss_l
        acc_s[1] += pos_ce

    @pl.when(b == _B)
    def _finalize():
        vn = vneg_s[...]
        num_pos = np_s[:, 0:1]
        k = jnp.minimum(num_pos * 3.0, float(_P - 1))

        vb = lax.bitcast_convert_type(vn, jnp.int32)
        prefix = jnp.zeros_like(vb[:, 0:1])
        for bit in range(30, -1, -1):
            cand = prefix | (1 << bit)
            c = jnp.sum((vb >= cand).astype(jnp.float32), axis=1,
                        keepdims=True)
            prefix = jnp.where(c >= k, cand, prefix)
        t = lax.bitcast_convert_type(prefix, jnp.float32)
        gt = vb > prefix
        c1 = jnp.sum(gt.astype(jnp.float32), axis=1, keepdims=True)
        sum_gt = jnp.sum(jnp.where(gt, vn, 0.0), axis=1, keepdims=True)
        topk = sum_gt + (k - c1) * t

        n = jnp.sum(num_pos)
        out_l_ref[...] = (acc_s[0] / n).reshape(1, 1)
        out_c_ref[...] = ((acc_s[1] + jnp.sum(topk)) / n).reshape(1, 1)


def kernel(loc_data, conf_data, priors, targets):
    conf_tr = jnp.transpose(conf_data, (0, 2, 1))
    loc_tr = jnp.transpose(loc_data, (0, 2, 1))
    priors_t = jnp.transpose(priors)

    loss_l, loss_c = pl.pallas_call(
        _loss_kernel,
        grid=(_B + 1,),
        in_specs=[
            pl.BlockSpec((1, _NOBJ, 5), lambda b: (jnp.minimum(b, _B - 1), 0, 0)),
            pl.BlockSpec((4, _P), lambda b: (0, 0)),
            pl.BlockSpec((1, 4, _P), lambda b: (jnp.minimum(b, _B - 1), 0, 0)),
            pl.BlockSpec((1, _C, _P), lambda b: (jnp.minimum(b, _B - 1), 0, 0)),
        ],
        out_specs=[
            pl.BlockSpec((1, 1), lambda b: (0, 0)),
            pl.BlockSpec((1, 1), lambda b: (0, 0)),
        ],
        out_shape=[
            jax.ShapeDtypeStruct((1, 1), jnp.float32),
            jax.ShapeDtypeStruct((1, 1), jnp.float32),
        ],
        scratch_shapes=[
            pltpu.VMEM((_B, _P), jnp.float32),
            pltpu.VMEM((_B, 128), jnp.float32),
            pltpu.SMEM((2,), jnp.float32),
        ],
    )(targets, priors_t, loc_tr, conf_tr)
    return loss_l[0, 0], loss_c[0, 0]
